# baseline (speedup 1.0000x reference)
import functools

import jax
import jax.numpy as jnp
from jax import lax
from jax.experimental import pallas as pl
from jax.experimental.pallas import tpu as pltpu
from jax.experimental.pallas import tpu_sc as plsc

_L = 16
_BLK = 128

_info = plsc.get_sparse_core_info()
_NW = _info.num_cores * _info.num_subcores

_mesh = plsc.VectorSubcoreMesh(core_axis_name="c", subcore_axis_name="s")
_params = pltpu.CompilerParams(
    use_tc_tiling_on_sc=True, needs_layout_passes=False)


def _wid():
    return lax.axis_index("s") * _info.num_cores + lax.axis_index("c")


def _transpose_block(src, dst, width):
    dvecs = [lax.iota(jnp.int32, _L) + g * _L for g in range(4)]
    zero = lax.iota(jnp.int32, _L) * 0

    @plsc.parallel_loop(0, width // 2, unroll=8)
    def body(np_):
        for par in range(2):
            nvec = zero + (np_ * 2 + par)
            for g in range(4):
                v = plsc.load_gather(src, [dvecs[g], nvec])
                dst[np_, pl.ds(par * 64 + g * _L, _L)] = v


_CW = 4 * _BLK


def _make_repack(n_nodes, n_rel, d):
    n_chunks = n_nodes // _CW
    n_tail = n_nodes % _CW
    assert n_tail < _BLK
    per_w = n_chunks // _NW
    rem = n_chunks % _NW

    @functools.partial(
        pl.kernel,
        mesh=_mesh,
        compiler_params=_params,
        out_type=(
            jax.ShapeDtypeStruct((n_nodes // 2, 2 * d), jnp.float32),
            jax.ShapeDtypeStruct((n_rel // 2, 2 * d), jnp.float32),
        ),
        scratch_types=[
            pltpu.VMEM((d, _CW), jnp.float32),
            pltpu.VMEM((d, _CW), jnp.float32),
            pltpu.VMEM((_CW // 2, 2 * d), jnp.float32),
            pltpu.VMEM((d, _BLK), jnp.float32),
            pltpu.VMEM((d, n_nodes % _CW or _L), jnp.float32),
            pltpu.VMEM((d, n_rel % _BLK or _L), jnp.float32),
            pltpu.SemaphoreType.DMA,
            pltpu.SemaphoreType.DMA,
            pltpu.SemaphoreType.DMA,
        ],
    )
    def repack(head_hbm, rel_hbm, tail_hbm, node_t, rel_t,
               node_p, rel_p, in0, in1, ob, rblk, ntail, rtail,
               si0, si1, so):
        w = _wid()
        ins = (in0, in1)
        sis = (si0, si1)

        nchk = per_w + jnp.where(w < rem, 1, 0)

        def start_in(k, ph):
            cid = k * _NW + w
            pltpu.async_copy(
                node_t.at[pl.ds(0, d), pl.ds(cid * _CW, _CW)],
                ins[ph], sis[ph])

        @pl.when(nchk > 0)
        def _():
            start_in(0, 0)

        def step(k, _):
            for ph in range(2):
                kk = k * 2 + ph

                @pl.when(kk < nchk)
                def _():
                    @pl.when(kk + 1 < nchk)
                    def _():
                        start_in(kk + 1, 1 - ph)
                    pltpu.make_async_copy(
                        node_t.at[pl.ds(0, d), pl.ds(0, _CW)],
                        ins[ph], sis[ph]).wait()
                    @pl.when(kk >= 1)
                    def _():
                        pltpu.make_async_copy(
                            ob, node_p.at[pl.ds(0, _CW // 2)], so).wait()
                    _transpose_block(ins[ph], ob, _CW)
                    cid = kk * _NW + w
                    pltpu.async_copy(
                        ob, node_p.at[pl.ds(cid * (_CW // 2), _CW // 2)],
                        so)
            return 0

        lax.fori_loop(0, (per_w + 2) // 2, step, 0)

        @pl.when(nchk > 0)
        def _():
            pltpu.make_async_copy(
                ob, node_p.at[pl.ds(0, _CW // 2)], so).wait()

        if n_tail:
            @pl.when(w == 0)
            def _():
                pltpu.sync_copy(
                    node_t.at[pl.ds(0, d), pl.ds(n_chunks * _CW, n_tail)],
                    ntail)
                _transpose_block(ntail, ob, n_tail)
                pltpu.sync_copy(
                    ob.at[pl.ds(0, n_tail // 2)],
                    node_p.at[pl.ds(n_chunks * (_CW // 2), n_tail // 2)])

        r_full = n_rel // _BLK
        r_tail = n_rel % _BLK

        @pl.when(w == 1)
        def _():
            def rstep(b, _):
                pltpu.sync_copy(
                    rel_t.at[pl.ds(0, d), pl.ds(b * _BLK, _BLK)], rblk)
                _transpose_block(rblk, ob, _BLK)
                pltpu.sync_copy(
                    ob.at[pl.ds(0, _BLK // 2)],
                    rel_p.at[pl.ds(b * (_BLK // 2), _BLK // 2)])
                return 0

            lax.fori_loop(0, r_full, rstep, 0)
            if r_tail:
                pltpu.sync_copy(
                    rel_t.at[pl.ds(0, d), pl.ds(r_full * _BLK, r_tail)],
                    rtail)
                _transpose_block(rtail, ob, r_tail)
                pltpu.sync_copy(
                    ob.at[pl.ds(0, r_tail // 2)],
                    rel_p.at[pl.ds(r_full * (_BLK // 2), r_tail // 2)])

    return repack


def _make_gather(n_nodes, n_rel, d, b):
    b_per_w = b // _NW
    n_groups = b_per_w // _L
    n_chunks = b_per_w // _BLK

    @functools.partial(
        pl.kernel,
        mesh=_mesh,
        compiler_params=_params,
        out_type=(
            jax.ShapeDtypeStruct((d, b), jnp.float32),
            jax.ShapeDtypeStruct((d, b), jnp.float32),
            jax.ShapeDtypeStruct((d, b), jnp.float32),
        ),
        scratch_types=[
            pltpu.VMEM((b_per_w,), jnp.int32),
            pltpu.VMEM((b_per_w,), jnp.int32),
            pltpu.VMEM((b_per_w,), jnp.int32),
            pltpu.VMEM((b_per_w,), jnp.int32),
            pltpu.VMEM((b_per_w,), jnp.int32),
            pltpu.VMEM((b_per_w,), jnp.int32),
            pltpu.VMEM((b_per_w, 2 * d), jnp.float32),
            pltpu.VMEM((d, b_per_w), jnp.float32),
            pltpu.SemaphoreType.DMA,
            pltpu.SemaphoreType.DMA,
        ],
    )
    def gather(head_hbm, rel_hbm, tail_hbm, node_p, rel_p,
               head_out, rel_out, tail_out,
               hidx, tidx, ridx, hq, tq, rq,
               pairs, colout, sem_g, sem_w):
        w = _wid()
        base = w * b_per_w
        sl = pl.ds(base, b_per_w)

        pltpu.sync_copy(head_hbm.at[sl], hidx)
        pltpu.sync_copy(tail_hbm.at[sl], tidx)
        pltpu.sync_copy(rel_hbm.at[sl], ridx)

        def compute_q(g, _):
            s = pl.ds(g * _L, _L)
            hq[s] = lax.shift_right_logical(hidx[s], 1)
            tq[s] = lax.shift_right_logical(tidx[s], 1)
            rq[s] = lax.shift_right_logical(ridx[s], 1)
            return 0

        lax.fori_loop(0, n_groups, compute_q, 0)

        first = [True]
        for tab, q, idx, out_hbm in (
            (node_p, hq, hidx, head_out),
            (rel_p, rq, ridx, rel_out),
            (node_p, tq, tidx, tail_out),
        ):
            gathers = [
                pltpu.async_copy(
                    tab.at[q.at[pl.ds(c * _BLK, _BLK)]],
                    pairs.at[pl.ds(c * _BLK, _BLK)], sem_g)
                for c in range(n_chunks)
            ]
            if not first[0]:
                pltpu.make_async_copy(
                    head_out.at[pl.ds(0, d), pl.ds(0, b_per_w)],
                    colout, sem_w).wait()
            first[0] = False
            for g in gathers:
                g.wait()

            dvecs = [lax.iota(jnp.int32, _L) + cc * _L
                     for cc in range(d // _L)]
            zero = lax.iota(jnp.int32, _L) * 0

            @plsc.parallel_loop(0, n_groups, unroll=2)
            def select(g, idx=idx):
                iv = idx[pl.ds(g * _L, _L)]
                offv = lax.shift_left(
                    lax.bitwise_and(iv, jnp.int32(1)), 6)
                for jj in range(_L):
                    off = offv[jj]
                    row = g * _L + jj
                    bvec = zero + row
                    for cc in range(d // _L):
                        v = pairs[row, pl.ds(off + cc * _L, _L)]
                        plsc.store_scatter(colout, [dvecs[cc], bvec], v)
            pltpu.async_copy(colout, out_hbm.at[pl.ds(0, d), sl], sem_w)
        pltpu.make_async_copy(
            head_out.at[pl.ds(0, d), pl.ds(0, b_per_w)], colout,
            sem_w).wait()

    return gather


def kernel(head_index, rel_type, tail_index, node_emb, rel_emb):
    B = head_index.shape[0]
    N, D = node_emb.shape
    R = rel_emb.shape[0]
    assert D == 64 and B % (_NW * _BLK) == 0

    node_t = node_emb.T
    rel_t = rel_emb.T

    node_p, rel_p = _make_repack(N, R, D)(
        head_index, rel_type, tail_index, node_t, rel_t)
    h, r, t = _make_gather(N, R, D, B)(
        head_index, rel_type, tail_index, node_p, rel_p)
    return (h.T, r.T, t.T)


# speedup vs baseline: 2.6450x; 2.6450x over previous
import functools

import jax
import jax.numpy as jnp
from jax import lax
from jax.experimental import pallas as pl
from jax.experimental.pallas import tpu as pltpu
from jax.experimental.pallas import tpu_sc as plsc

_L = 16
_BLK = 128

_info = plsc.get_sparse_core_info()
_NW = _info.num_cores * _info.num_subcores

_mesh = plsc.VectorSubcoreMesh(core_axis_name="c", subcore_axis_name="s")
_params = pltpu.CompilerParams(
    use_tc_tiling_on_sc=True, needs_layout_passes=False)


def _wid():
    return lax.axis_index("s") * _info.num_cores + lax.axis_index("c")


def _transpose_block(src, dst, width):
    dvecs = [lax.iota(jnp.int32, _L) + g * _L for g in range(4)]
    zero = lax.iota(jnp.int32, _L) * 0

    @plsc.parallel_loop(0, width // 2, unroll=8)
    def body(np_):
        for par in range(2):
            nvec = zero + (np_ * 2 + par)
            for g in range(4):
                v = plsc.load_gather(src, [dvecs[g], nvec])
                dst[np_, pl.ds(par * 64 + g * _L, _L)] = v


def _transpose_diag(src, dst, width):
    iota = lax.iota(jnp.int32, _L)
    dvecs = [iota + d0 for d0 in range(0, 64, _L)]

    @plsc.parallel_loop(0, width // _L, unroll=1)
    def body(g):
        n0 = g * _L
        n0h = lax.shift_right_logical(n0, 1)
        m = iota
        for j in range(_L):
            nvec = m + n0
            rowvec = lax.shift_right_logical(m, 1) + n0h
            colbase = lax.shift_left(
                jnp.bitwise_and(m, jnp.int32(1)), 6) + iota
            for gi in range(4):
                v = plsc.load_gather(src, [dvecs[gi], nvec])
                plsc.store_scatter(dst, [rowvec, colbase + gi * _L], v)
            if j + 1 < _L:
                m = jnp.bitwise_and(m + 1, jnp.int32(15))


_CW = 4 * _BLK


def _make_repack(n_nodes, n_rel, d):
    n_chunks = n_nodes // _CW
    n_tail = n_nodes % _CW
    assert n_tail < _BLK
    per_w = n_chunks // _NW
    rem = n_chunks % _NW

    @functools.partial(
        pl.kernel,
        mesh=_mesh,
        compiler_params=_params,
        out_type=(
            jax.ShapeDtypeStruct((n_nodes // 2, 2 * d), jnp.float32),
            jax.ShapeDtypeStruct((n_rel // 2, 2 * d), jnp.float32),
        ),
        scratch_types=[
            pltpu.VMEM((d, _CW), jnp.float32),
            pltpu.VMEM((d, _CW), jnp.float32),
            pltpu.VMEM((_CW // 2, 2 * d), jnp.float32),
            pltpu.VMEM((d, _BLK), jnp.float32),
            pltpu.VMEM((d, n_nodes % _CW or _L), jnp.float32),
            pltpu.VMEM((d, n_rel % _BLK or _L), jnp.float32),
            pltpu.SemaphoreType.DMA,
            pltpu.SemaphoreType.DMA,
            pltpu.SemaphoreType.DMA,
        ],
    )
    def repack(head_hbm, rel_hbm, tail_hbm, node_t, rel_t,
               node_p, rel_p, in0, in1, ob, rblk, ntail, rtail,
               si0, si1, so):
        w = _wid()
        ins = (in0, in1)
        sis = (si0, si1)

        nchk = per_w + jnp.where(w < rem, 1, 0)

        def start_in(k, ph):
            cid = k * _NW + w
            pltpu.async_copy(
                node_t.at[pl.ds(0, d), pl.ds(cid * _CW, _CW)],
                ins[ph], sis[ph])

        @pl.when(nchk > 0)
        def _():
            start_in(0, 0)

        def step(k, _):
            for ph in range(2):
                kk = k * 2 + ph

                @pl.when(kk < nchk)
                def _():
                    @pl.when(kk + 1 < nchk)
                    def _():
                        start_in(kk + 1, 1 - ph)
                    pltpu.make_async_copy(
                        node_t.at[pl.ds(0, d), pl.ds(0, _CW)],
                        ins[ph], sis[ph]).wait()
                    @pl.when(kk >= 1)
                    def _():
                        pltpu.make_async_copy(
                            ob, node_p.at[pl.ds(0, _CW // 2)], so).wait()
                    _transpose_diag(ins[ph], ob, _CW)
                    cid = kk * _NW + w
                    pltpu.async_copy(
                        ob, node_p.at[pl.ds(cid * (_CW // 2), _CW // 2)],
                        so)
            return 0

        lax.fori_loop(0, (per_w + 2) // 2, step, 0)

        @pl.when(nchk > 0)
        def _():
            pltpu.make_async_copy(
                ob, node_p.at[pl.ds(0, _CW // 2)], so).wait()

        if n_tail:
            @pl.when(w == 0)
            def _():
                pltpu.sync_copy(
                    node_t.at[pl.ds(0, d), pl.ds(n_chunks * _CW, n_tail)],
                    ntail)
                _transpose_block(ntail, ob, n_tail)
                pltpu.sync_copy(
                    ob.at[pl.ds(0, n_tail // 2)],
                    node_p.at[pl.ds(n_chunks * (_CW // 2), n_tail // 2)])

        r_full = n_rel // _BLK
        r_tail = n_rel % _BLK

        @pl.when(w == 1)
        def _():
            def rstep(b, _):
                pltpu.sync_copy(
                    rel_t.at[pl.ds(0, d), pl.ds(b * _BLK, _BLK)], rblk)
                _transpose_diag(rblk, ob, _BLK)
                pltpu.sync_copy(
                    ob.at[pl.ds(0, _BLK // 2)],
                    rel_p.at[pl.ds(b * (_BLK // 2), _BLK // 2)])
                return 0

            lax.fori_loop(0, r_full, rstep, 0)
            if r_tail:
                pltpu.sync_copy(
                    rel_t.at[pl.ds(0, d), pl.ds(r_full * _BLK, r_tail)],
                    rtail)
                _transpose_block(rtail, ob, r_tail)
                pltpu.sync_copy(
                    ob.at[pl.ds(0, r_tail // 2)],
                    rel_p.at[pl.ds(r_full * (_BLK // 2), r_tail // 2)])

    return repack


def _make_gather(n_nodes, n_rel, d, b):
    b_per_w = b // _NW
    n_groups = b_per_w // _L
    n_chunks = b_per_w // _BLK

    @functools.partial(
        pl.kernel,
        mesh=_mesh,
        compiler_params=_params,
        out_type=(
            jax.ShapeDtypeStruct((d, b), jnp.float32),
            jax.ShapeDtypeStruct((d, b), jnp.float32),
            jax.ShapeDtypeStruct((d, b), jnp.float32),
        ),
        scratch_types=[
            pltpu.VMEM((b_per_w,), jnp.int32),
            pltpu.VMEM((b_per_w,), jnp.int32),
            pltpu.VMEM((b_per_w,), jnp.int32),
            pltpu.VMEM((b_per_w,), jnp.int32),
            pltpu.VMEM((b_per_w,), jnp.int32),
            pltpu.VMEM((b_per_w,), jnp.int32),
            pltpu.VMEM((b_per_w, 2 * d), jnp.float32),
            pltpu.VMEM((d, b_per_w), jnp.float32),
            pltpu.SemaphoreType.DMA,
            pltpu.SemaphoreType.DMA,
        ],
    )
    def gather(head_hbm, rel_hbm, tail_hbm, node_p, rel_p,
               head_out, rel_out, tail_out,
               hidx, tidx, ridx, hq, tq, rq,
               pairs, colout, sem_g, sem_w):
        w = _wid()
        base = w * b_per_w
        sl = pl.ds(base, b_per_w)

        pltpu.sync_copy(head_hbm.at[sl], hidx)
        pltpu.sync_copy(tail_hbm.at[sl], tidx)
        pltpu.sync_copy(rel_hbm.at[sl], ridx)

        def compute_q(g, _):
            s = pl.ds(g * _L, _L)
            hq[s] = lax.shift_right_logical(hidx[s], 1)
            tq[s] = lax.shift_right_logical(tidx[s], 1)
            rq[s] = lax.shift_right_logical(ridx[s], 1)
            return 0

        lax.fori_loop(0, n_groups, compute_q, 0)

        first = [True]
        for tab, q, idx, out_hbm in (
            (node_p, hq, hidx, head_out),
            (rel_p, rq, ridx, rel_out),
            (node_p, tq, tidx, tail_out),
        ):
            gathers = [
                pltpu.async_copy(
                    tab.at[q.at[pl.ds(c * _BLK, _BLK)]],
                    pairs.at[pl.ds(c * _BLK, _BLK)], sem_g)
                for c in range(n_chunks)
            ]
            if not first[0]:
                pltpu.make_async_copy(
                    head_out.at[pl.ds(0, d), pl.ds(0, b_per_w)],
                    colout, sem_w).wait()
            first[0] = False
            for g in gathers:
                g.wait()

            dvecs = [lax.iota(jnp.int32, _L) + cc * _L
                     for cc in range(d // _L)]
            zero = lax.iota(jnp.int32, _L) * 0

            @plsc.parallel_loop(0, n_groups, unroll=2)
            def select(g, idx=idx):
                iv = idx[pl.ds(g * _L, _L)]
                offv = lax.shift_left(
                    lax.bitwise_and(iv, jnp.int32(1)), 6)
                for jj in range(_L):
                    off = offv[jj]
                    row = g * _L + jj
                    bvec = zero + row
                    for cc in range(d // _L):
                        v = pairs[row, pl.ds(off + cc * _L, _L)]
                        plsc.store_scatter(colout, [dvecs[cc], bvec], v)
            pltpu.async_copy(colout, out_hbm.at[pl.ds(0, d), sl], sem_w)
        pltpu.make_async_copy(
            head_out.at[pl.ds(0, d), pl.ds(0, b_per_w)], colout,
            sem_w).wait()

    return gather


def kernel(head_index, rel_type, tail_index, node_emb, rel_emb):
    B = head_index.shape[0]
    N, D = node_emb.shape
    R = rel_emb.shape[0]
    assert D == 64 and B % (_NW * _BLK) == 0

    node_t = node_emb.T
    rel_t = rel_emb.T

    node_p, rel_p = _make_repack(N, R, D)(
        head_index, rel_type, tail_index, node_t, rel_t)
    h, r, t = _make_gather(N, R, D, B)(
        head_index, rel_type, tail_index, node_p, rel_p)
    return (h.T, r.T, t.T)


# speedup vs baseline: 2.8424x; 1.0746x over previous
import functools

import jax
import jax.numpy as jnp
from jax import lax
from jax.experimental import pallas as pl
from jax.experimental.pallas import tpu as pltpu
from jax.experimental.pallas import tpu_sc as plsc

_L = 16
_BLK = 128

_info = plsc.get_sparse_core_info()
_NW = _info.num_cores * _info.num_subcores

_mesh = plsc.VectorSubcoreMesh(core_axis_name="c", subcore_axis_name="s")
_params = pltpu.CompilerParams(
    use_tc_tiling_on_sc=True, needs_layout_passes=False)


def _wid():
    return lax.axis_index("s") * _info.num_cores + lax.axis_index("c")


def _transpose_block(src, dst, width):
    dvecs = [lax.iota(jnp.int32, _L) + g * _L for g in range(4)]
    zero = lax.iota(jnp.int32, _L) * 0

    @plsc.parallel_loop(0, width // 2, unroll=8)
    def body(np_):
        for par in range(2):
            nvec = zero + (np_ * 2 + par)
            for g in range(4):
                v = plsc.load_gather(src, [dvecs[g], nvec])
                dst[np_, pl.ds(par * 64 + g * _L, _L)] = v


def _transpose_diag(src, dst, width):
    iota = lax.iota(jnp.int32, _L)
    dvecs = [iota + d0 for d0 in range(0, 64, _L)]

    @plsc.parallel_loop(0, width // _L, unroll=1)
    def body(g):
        n0 = g * _L
        n0h = lax.shift_right_logical(n0, 1)
        m = iota
        for j in range(_L):
            nvec = m + n0
            rowvec = lax.shift_right_logical(m, 1) + n0h
            colbase = lax.shift_left(
                jnp.bitwise_and(m, jnp.int32(1)), 6) + iota
            for gi in range(4):
                v = plsc.load_gather(src, [dvecs[gi], nvec])
                plsc.store_scatter(dst, [rowvec, colbase + gi * _L], v)
            if j + 1 < _L:
                m = jnp.bitwise_and(m + 1, jnp.int32(15))


_CW = 4 * _BLK


def _make_repack(n_nodes, n_rel, d):
    n_chunks = n_nodes // _CW
    n_tail = n_nodes % _CW
    assert n_tail < _BLK
    per_w = n_chunks // _NW
    rem = n_chunks % _NW

    @functools.partial(
        pl.kernel,
        mesh=_mesh,
        compiler_params=_params,
        out_type=(
            jax.ShapeDtypeStruct((n_nodes // 2, 2 * d), jnp.float32),
            jax.ShapeDtypeStruct((n_rel // 2, 2 * d), jnp.float32),
        ),
        scratch_types=[
            pltpu.VMEM((d, _CW), jnp.float32),
            pltpu.VMEM((d, _CW), jnp.float32),
            pltpu.VMEM((_CW // 2, 2 * d), jnp.float32),
            pltpu.VMEM((d, _BLK), jnp.float32),
            pltpu.VMEM((d, n_nodes % _CW or _L), jnp.float32),
            pltpu.VMEM((d, n_rel % _BLK or _L), jnp.float32),
            pltpu.SemaphoreType.DMA,
            pltpu.SemaphoreType.DMA,
            pltpu.SemaphoreType.DMA,
        ],
    )
    def repack(head_hbm, rel_hbm, tail_hbm, node_t, rel_t,
               node_p, rel_p, in0, in1, ob, rblk, ntail, rtail,
               si0, si1, so):
        w = _wid()
        ins = (in0, in1)
        sis = (si0, si1)

        nchk = per_w + jnp.where(w < rem, 1, 0)

        def start_in(k, ph):
            cid = k * _NW + w
            pltpu.async_copy(
                node_t.at[pl.ds(0, d), pl.ds(cid * _CW, _CW)],
                ins[ph], sis[ph])

        @pl.when(nchk > 0)
        def _():
            start_in(0, 0)

        def step(k, _):
            for ph in range(2):
                kk = k * 2 + ph

                @pl.when(kk < nchk)
                def _():
                    @pl.when(kk + 1 < nchk)
                    def _():
                        start_in(kk + 1, 1 - ph)
                    pltpu.make_async_copy(
                        node_t.at[pl.ds(0, d), pl.ds(0, _CW)],
                        ins[ph], sis[ph]).wait()
                    @pl.when(kk >= 1)
                    def _():
                        pltpu.make_async_copy(
                            ob, node_p.at[pl.ds(0, _CW // 2)], so).wait()
                    _transpose_diag(ins[ph], ob, _CW)
                    cid = kk * _NW + w
                    pltpu.async_copy(
                        ob, node_p.at[pl.ds(cid * (_CW // 2), _CW // 2)],
                        so)
            return 0

        lax.fori_loop(0, (per_w + 2) // 2, step, 0)

        @pl.when(nchk > 0)
        def _():
            pltpu.make_async_copy(
                ob, node_p.at[pl.ds(0, _CW // 2)], so).wait()

        if n_tail:
            @pl.when(w == 0)
            def _():
                pltpu.sync_copy(
                    node_t.at[pl.ds(0, d), pl.ds(n_chunks * _CW, n_tail)],
                    ntail)
                _transpose_block(ntail, ob, n_tail)
                pltpu.sync_copy(
                    ob.at[pl.ds(0, n_tail // 2)],
                    node_p.at[pl.ds(n_chunks * (_CW // 2), n_tail // 2)])

        r_full = n_rel // _BLK
        r_tail = n_rel % _BLK

        @pl.when(w == 1)
        def _():
            def rstep(b, _):
                pltpu.sync_copy(
                    rel_t.at[pl.ds(0, d), pl.ds(b * _BLK, _BLK)], rblk)
                _transpose_diag(rblk, ob, _BLK)
                pltpu.sync_copy(
                    ob.at[pl.ds(0, _BLK // 2)],
                    rel_p.at[pl.ds(b * (_BLK // 2), _BLK // 2)])
                return 0

            lax.fori_loop(0, r_full, rstep, 0)
            if r_tail:
                pltpu.sync_copy(
                    rel_t.at[pl.ds(0, d), pl.ds(r_full * _BLK, r_tail)],
                    rtail)
                _transpose_block(rtail, ob, r_tail)
                pltpu.sync_copy(
                    ob.at[pl.ds(0, r_tail // 2)],
                    rel_p.at[pl.ds(r_full * (_BLK // 2), r_tail // 2)])

    return repack


def _make_gather(n_nodes, n_rel, d, b):
    b_per_w = b // _NW
    n_groups = b_per_w // _L
    n_chunks = b_per_w // _BLK

    @functools.partial(
        pl.kernel,
        mesh=_mesh,
        compiler_params=_params,
        out_type=(
            jax.ShapeDtypeStruct((d, b), jnp.float32),
            jax.ShapeDtypeStruct((d, b), jnp.float32),
            jax.ShapeDtypeStruct((d, b), jnp.float32),
        ),
        scratch_types=[
            pltpu.VMEM((b_per_w,), jnp.int32),
            pltpu.VMEM((b_per_w,), jnp.int32),
            pltpu.VMEM((b_per_w,), jnp.int32),
            pltpu.VMEM((b_per_w,), jnp.int32),
            pltpu.VMEM((b_per_w,), jnp.int32),
            pltpu.VMEM((b_per_w,), jnp.int32),
            pltpu.VMEM((b_per_w, 2 * d), jnp.float32),
            pltpu.VMEM((d, b_per_w), jnp.float32),
            pltpu.SemaphoreType.DMA,
            pltpu.SemaphoreType.DMA,
        ],
    )
    def gather(head_hbm, rel_hbm, tail_hbm, node_p, rel_p,
               head_out, rel_out, tail_out,
               hidx, tidx, ridx, hq, tq, rq,
               pairs, colout, sem_g, sem_w):
        w = _wid()
        base = w * b_per_w
        sl = pl.ds(base, b_per_w)

        pltpu.sync_copy(head_hbm.at[sl], hidx)
        pltpu.sync_copy(tail_hbm.at[sl], tidx)
        pltpu.sync_copy(rel_hbm.at[sl], ridx)

        def compute_q(g, _):
            s = pl.ds(g * _L, _L)
            hq[s] = lax.shift_right_logical(hidx[s], 1)
            tq[s] = lax.shift_right_logical(tidx[s], 1)
            rq[s] = lax.shift_right_logical(ridx[s], 1)
            return 0

        lax.fori_loop(0, n_groups, compute_q, 0)

        first = [True]
        for tab, q, idx, out_hbm in (
            (node_p, hq, hidx, head_out),
            (rel_p, rq, ridx, rel_out),
            (node_p, tq, tidx, tail_out),
        ):
            gathers = [
                pltpu.async_copy(
                    tab.at[q.at[pl.ds(c * _BLK, _BLK)]],
                    pairs.at[pl.ds(c * _BLK, _BLK)], sem_g)
                for c in range(n_chunks)
            ]
            if not first[0]:
                pltpu.make_async_copy(
                    head_out.at[pl.ds(0, d), pl.ds(0, b_per_w)],
                    colout, sem_w).wait()
            first[0] = False
            for g in gathers:
                g.wait()

            iota = lax.iota(jnp.int32, _L)

            @plsc.parallel_loop(0, n_groups, unroll=1)
            def select(g, idx=idx):
                r0 = g * _L
                rowvec = iota + r0
                iv = idx[pl.ds(r0, _L)]
                offv = lax.shift_left(
                    lax.bitwise_and(iv, jnp.int32(1)), 6)
                m = iota
                for j in range(_L):
                    for d0 in range(0, d, _L):
                        fv = m + d0
                        v = plsc.load_gather(pairs, [rowvec, offv + fv])
                        plsc.store_scatter(colout, [fv, rowvec], v)
                    if j + 1 < _L:
                        m = jnp.bitwise_and(m + 1, jnp.int32(15))
            pltpu.async_copy(colout, out_hbm.at[pl.ds(0, d), sl], sem_w)
        pltpu.make_async_copy(
            head_out.at[pl.ds(0, d), pl.ds(0, b_per_w)], colout,
            sem_w).wait()

    return gather


def kernel(head_index, rel_type, tail_index, node_emb, rel_emb):
    B = head_index.shape[0]
    N, D = node_emb.shape
    R = rel_emb.shape[0]
    assert D == 64 and B % (_NW * _BLK) == 0

    node_t = node_emb.T
    rel_t = rel_emb.T

    node_p, rel_p = _make_repack(N, R, D)(
        head_index, rel_type, tail_index, node_t, rel_t)
    h, r, t = _make_gather(N, R, D, B)(
        head_index, rel_type, tail_index, node_p, rel_p)
    return (h.T, r.T, t.T)


# speedup vs baseline: 2.8486x; 1.0022x over previous
import functools

import jax
import jax.numpy as jnp
from jax import lax
from jax.experimental import pallas as pl
from jax.experimental.pallas import tpu as pltpu
from jax.experimental.pallas import tpu_sc as plsc

_L = 16
_BLK = 128

_info = plsc.get_sparse_core_info()
_NW = _info.num_cores * _info.num_subcores

_mesh = plsc.VectorSubcoreMesh(core_axis_name="c", subcore_axis_name="s")
_params = pltpu.CompilerParams(
    use_tc_tiling_on_sc=True, needs_layout_passes=False)


def _wid():
    return lax.axis_index("s") * _info.num_cores + lax.axis_index("c")


def _transpose_block(src, dst, width):
    dvecs = [lax.iota(jnp.int32, _L) + g * _L for g in range(4)]
    zero = lax.iota(jnp.int32, _L) * 0

    @plsc.parallel_loop(0, width // 2, unroll=8)
    def body(np_):
        for par in range(2):
            nvec = zero + (np_ * 2 + par)
            for g in range(4):
                v = plsc.load_gather(src, [dvecs[g], nvec])
                dst[np_, pl.ds(par * 64 + g * _L, _L)] = v


def _transpose_diag(src, dst, width):
    iota = lax.iota(jnp.int32, _L)
    dvecs = [iota + d0 for d0 in range(0, 64, _L)]

    @plsc.parallel_loop(0, width // _L, unroll=1)
    def body(g):
        n0 = g * _L
        n0h = lax.shift_right_logical(n0, 1)
        for j in range(_L):
            m = jnp.bitwise_and(iota + j, jnp.int32(15))
            nvec = m + n0
            rowvec = lax.shift_right_logical(m, 1) + n0h
            colbase = lax.shift_left(
                jnp.bitwise_and(m, jnp.int32(1)), 6) + iota
            for gi in range(4):
                v = plsc.load_gather(src, [dvecs[gi], nvec])
                plsc.store_scatter(dst, [rowvec, colbase + gi * _L], v)


_CW = 4 * _BLK


def _make_repack(n_nodes, n_rel, d):
    n_chunks = n_nodes // _CW
    n_tail = n_nodes % _CW
    assert n_tail < _BLK
    per_w = n_chunks // _NW
    rem = n_chunks % _NW

    @functools.partial(
        pl.kernel,
        mesh=_mesh,
        compiler_params=_params,
        out_type=(
            jax.ShapeDtypeStruct((n_nodes // 2, 2 * d), jnp.float32),
            jax.ShapeDtypeStruct((n_rel // 2, 2 * d), jnp.float32),
        ),
        scratch_types=[
            pltpu.VMEM((d, _CW), jnp.float32),
            pltpu.VMEM((d, _CW), jnp.float32),
            pltpu.VMEM((_CW // 2, 2 * d), jnp.float32),
            pltpu.VMEM((d, _BLK), jnp.float32),
            pltpu.VMEM((d, n_nodes % _CW or _L), jnp.float32),
            pltpu.VMEM((d, n_rel % _BLK or _L), jnp.float32),
            pltpu.SemaphoreType.DMA,
            pltpu.SemaphoreType.DMA,
            pltpu.SemaphoreType.DMA,
        ],
    )
    def repack(head_hbm, rel_hbm, tail_hbm, node_t, rel_t,
               node_p, rel_p, in0, in1, ob, rblk, ntail, rtail,
               si0, si1, so):
        w = _wid()
        ins = (in0, in1)
        sis = (si0, si1)

        nchk = per_w + jnp.where(w < rem, 1, 0)

        def start_in(k, ph):
            cid = k * _NW + w
            pltpu.async_copy(
                node_t.at[pl.ds(0, d), pl.ds(cid * _CW, _CW)],
                ins[ph], sis[ph])

        @pl.when(nchk > 0)
        def _():
            start_in(0, 0)

        def step(k, _):
            for ph in range(2):
                kk = k * 2 + ph

                @pl.when(kk < nchk)
                def _():
                    @pl.when(kk + 1 < nchk)
                    def _():
                        start_in(kk + 1, 1 - ph)
                    pltpu.make_async_copy(
                        node_t.at[pl.ds(0, d), pl.ds(0, _CW)],
                        ins[ph], sis[ph]).wait()
                    @pl.when(kk >= 1)
                    def _():
                        pltpu.make_async_copy(
                            ob, node_p.at[pl.ds(0, _CW // 2)], so).wait()
                    _transpose_diag(ins[ph], ob, _CW)
                    cid = kk * _NW + w
                    pltpu.async_copy(
                        ob, node_p.at[pl.ds(cid * (_CW // 2), _CW // 2)],
                        so)
            return 0

        lax.fori_loop(0, (per_w + 2) // 2, step, 0)

        @pl.when(nchk > 0)
        def _():
            pltpu.make_async_copy(
                ob, node_p.at[pl.ds(0, _CW // 2)], so).wait()

        if n_tail:
            @pl.when(w == 0)
            def _():
                pltpu.sync_copy(
                    node_t.at[pl.ds(0, d), pl.ds(n_chunks * _CW, n_tail)],
                    ntail)
                _transpose_block(ntail, ob, n_tail)
                pltpu.sync_copy(
                    ob.at[pl.ds(0, n_tail // 2)],
                    node_p.at[pl.ds(n_chunks * (_CW // 2), n_tail // 2)])

        r_full = n_rel // _BLK
        r_tail = n_rel % _BLK

        @pl.when(w == 1)
        def _():
            def rstep(b, _):
                pltpu.sync_copy(
                    rel_t.at[pl.ds(0, d), pl.ds(b * _BLK, _BLK)], rblk)
                _transpose_diag(rblk, ob, _BLK)
                pltpu.sync_copy(
                    ob.at[pl.ds(0, _BLK // 2)],
                    rel_p.at[pl.ds(b * (_BLK // 2), _BLK // 2)])
                return 0

            lax.fori_loop(0, r_full, rstep, 0)
            if r_tail:
                pltpu.sync_copy(
                    rel_t.at[pl.ds(0, d), pl.ds(r_full * _BLK, r_tail)],
                    rtail)
                _transpose_block(rtail, ob, r_tail)
                pltpu.sync_copy(
                    ob.at[pl.ds(0, r_tail // 2)],
                    rel_p.at[pl.ds(r_full * (_BLK // 2), r_tail // 2)])

    return repack


def _make_gather(n_nodes, n_rel, d, b):
    b_per_w = b // _NW
    n_groups = b_per_w // _L
    n_chunks = b_per_w // _BLK

    @functools.partial(
        pl.kernel,
        mesh=_mesh,
        compiler_params=_params,
        out_type=(
            jax.ShapeDtypeStruct((d, b), jnp.float32),
            jax.ShapeDtypeStruct((d, b), jnp.float32),
            jax.ShapeDtypeStruct((d, b), jnp.float32),
        ),
        scratch_types=[
            pltpu.VMEM((b_per_w,), jnp.int32),
            pltpu.VMEM((b_per_w,), jnp.int32),
            pltpu.VMEM((b_per_w,), jnp.int32),
            pltpu.VMEM((b_per_w,), jnp.int32),
            pltpu.VMEM((b_per_w,), jnp.int32),
            pltpu.VMEM((b_per_w,), jnp.int32),
            pltpu.VMEM((b_per_w, 2 * d), jnp.float32),
            pltpu.VMEM((d, b_per_w), jnp.float32),
            pltpu.SemaphoreType.DMA,
            pltpu.SemaphoreType.DMA,
        ],
    )
    def gather(head_hbm, rel_hbm, tail_hbm, node_p, rel_p,
               head_out, rel_out, tail_out,
               hidx, tidx, ridx, hq, tq, rq,
               pairs, colout, sem_g, sem_w):
        w = _wid()
        base = w * b_per_w
        sl = pl.ds(base, b_per_w)

        pltpu.sync_copy(head_hbm.at[sl], hidx)
        pltpu.sync_copy(tail_hbm.at[sl], tidx)
        pltpu.sync_copy(rel_hbm.at[sl], ridx)

        def compute_q(g, _):
            s = pl.ds(g * _L, _L)
            hq[s] = lax.shift_right_logical(hidx[s], 1)
            tq[s] = lax.shift_right_logical(tidx[s], 1)
            rq[s] = lax.shift_right_logical(ridx[s], 1)
            return 0

        lax.fori_loop(0, n_groups, compute_q, 0)

        first = [True]
        for tab, q, idx, out_hbm in (
            (node_p, hq, hidx, head_out),
            (rel_p, rq, ridx, rel_out),
            (node_p, tq, tidx, tail_out),
        ):
            gathers = [
                pltpu.async_copy(
                    tab.at[q.at[pl.ds(c * _BLK, _BLK)]],
                    pairs.at[pl.ds(c * _BLK, _BLK)], sem_g)
                for c in range(n_chunks)
            ]
            if not first[0]:
                pltpu.make_async_copy(
                    head_out.at[pl.ds(0, d), pl.ds(0, b_per_w)],
                    colout, sem_w).wait()
            first[0] = False
            for g in gathers:
                g.wait()

            iota = lax.iota(jnp.int32, _L)

            @plsc.parallel_loop(0, n_groups, unroll=1)
            def select(g, idx=idx):
                r0 = g * _L
                rowvec = iota + r0
                iv = idx[pl.ds(r0, _L)]
                offv = lax.shift_left(
                    lax.bitwise_and(iv, jnp.int32(1)), 6)
                for j in range(_L):
                    m = jnp.bitwise_and(iota + j, jnp.int32(15))
                    for d0 in range(0, d, _L):
                        fv = m + d0
                        v = plsc.load_gather(pairs, [rowvec, offv + fv])
                        plsc.store_scatter(colout, [fv, rowvec], v)
            pltpu.async_copy(colout, out_hbm.at[pl.ds(0, d), sl], sem_w)
        pltpu.make_async_copy(
            head_out.at[pl.ds(0, d), pl.ds(0, b_per_w)], colout,
            sem_w).wait()

    return gather


def kernel(head_index, rel_type, tail_index, node_emb, rel_emb):
    B = head_index.shape[0]
    N, D = node_emb.shape
    R = rel_emb.shape[0]
    assert D == 64 and B % (_NW * _BLK) == 0

    node_t = node_emb.T
    rel_t = rel_emb.T

    node_p, rel_p = _make_repack(N, R, D)(
        head_index, rel_type, tail_index, node_t, rel_t)
    h, r, t = _make_gather(N, R, D, B)(
        head_index, rel_type, tail_index, node_p, rel_p)
    return (h.T, r.T, t.T)


# speedup vs baseline: 2.9471x; 1.0346x over previous
import functools

import jax
import jax.numpy as jnp
from jax import lax
from jax.experimental import pallas as pl
from jax.experimental.pallas import tpu as pltpu
from jax.experimental.pallas import tpu_sc as plsc

_L = 16
_BLK = 128

_info = plsc.get_sparse_core_info()
_NW = _info.num_cores * _info.num_subcores

_mesh = plsc.VectorSubcoreMesh(core_axis_name="c", subcore_axis_name="s")
_params = pltpu.CompilerParams(
    use_tc_tiling_on_sc=True, needs_layout_passes=False)


def _wid():
    return lax.axis_index("s") * _info.num_cores + lax.axis_index("c")


def _transpose_block(src, dst, width):
    dvecs = [lax.iota(jnp.int32, _L) + g * _L for g in range(4)]
    zero = lax.iota(jnp.int32, _L) * 0

    @plsc.parallel_loop(0, width // 2, unroll=8)
    def body(np_):
        for par in range(2):
            nvec = zero + (np_ * 2 + par)
            for g in range(4):
                v = plsc.load_gather(src, [dvecs[g], nvec])
                dst[np_, pl.ds(par * 64 + g * _L, _L)] = v


def _transpose_diag(src, dst, width):
    iota = lax.iota(jnp.int32, _L)
    dvecs = [iota + d0 for d0 in range(0, 64, _L)]

    @plsc.parallel_loop(0, width // _L, unroll=1)
    def body(g):
        n0 = g * _L
        n0h = lax.shift_right_logical(n0, 1)
        for j in range(_L):
            m = jnp.bitwise_and(iota + j, jnp.int32(15))
            nvec = m + n0
            rowvec = lax.shift_right_logical(m, 1) + n0h
            colbase = lax.shift_left(
                jnp.bitwise_and(m, jnp.int32(1)), 6) + iota
            for gi in range(4):
                v = plsc.load_gather(src, [dvecs[gi], nvec])
                plsc.store_scatter(dst, [rowvec, colbase + gi * _L], v)


_CW = 2 * _BLK


def _make_repack(n_nodes, n_rel, d):
    n_chunks = n_nodes // _CW
    n_tail = n_nodes % _CW
    assert n_tail < _BLK
    per_w = n_chunks // _NW
    rem = n_chunks % _NW

    @functools.partial(
        pl.kernel,
        mesh=_mesh,
        compiler_params=_params,
        out_type=(
            jax.ShapeDtypeStruct((n_nodes // 2, 2 * d), jnp.float32),
            jax.ShapeDtypeStruct((n_rel // 2, 2 * d), jnp.float32),
        ),
        scratch_types=[
            pltpu.VMEM((d, _CW), jnp.float32),
            pltpu.VMEM((d, _CW), jnp.float32),
            pltpu.VMEM((_CW // 2, 2 * d), jnp.float32),
            pltpu.VMEM((_CW // 2, 2 * d), jnp.float32),
            pltpu.VMEM((d, _BLK), jnp.float32),
            pltpu.VMEM((d, n_nodes % _CW or _L), jnp.float32),
            pltpu.VMEM((d, n_rel % _BLK or _L), jnp.float32),
            pltpu.SemaphoreType.DMA,
            pltpu.SemaphoreType.DMA,
            pltpu.SemaphoreType.DMA,
            pltpu.SemaphoreType.DMA,
        ],
    )
    def repack(head_hbm, rel_hbm, tail_hbm, node_t, rel_t,
               node_p, rel_p, in0, in1, ob0, ob1, rblk, ntail, rtail,
               si0, si1, so0, so1):
        w = _wid()
        ins = (in0, in1)
        sis = (si0, si1)
        obs = (ob0, ob1)
        sos = (so0, so1)

        nchk = per_w + jnp.where(w < rem, 1, 0)

        def start_in(k, ph):
            cid = k * _NW + w
            pltpu.async_copy(
                node_t.at[pl.ds(0, d), pl.ds(cid * _CW, _CW)],
                ins[ph], sis[ph])

        @pl.when(nchk > 0)
        def _():
            start_in(0, 0)

        def step(k, _):
            for ph in range(2):
                kk = k * 2 + ph

                @pl.when(kk < nchk)
                def _():
                    @pl.when(kk + 1 < nchk)
                    def _():
                        start_in(kk + 1, 1 - ph)
                    pltpu.make_async_copy(
                        node_t.at[pl.ds(0, d), pl.ds(0, _CW)],
                        ins[ph], sis[ph]).wait()
                    @pl.when(kk >= 2)
                    def _():
                        pltpu.make_async_copy(
                            obs[ph], node_p.at[pl.ds(0, _CW // 2)],
                            sos[ph]).wait()
                    _transpose_diag(ins[ph], obs[ph], _CW)
                    cid = kk * _NW + w
                    pltpu.async_copy(
                        obs[ph],
                        node_p.at[pl.ds(cid * (_CW // 2), _CW // 2)],
                        sos[ph])
            return 0

        lax.fori_loop(0, (per_w + 2) // 2, step, 0)

        for ph in range(2):
            @pl.when(nchk > ph)
            def _(ph=ph):
                pltpu.make_async_copy(
                    obs[ph], node_p.at[pl.ds(0, _CW // 2)],
                    sos[ph]).wait()

        if n_tail:
            @pl.when(w == 0)
            def _():
                pltpu.sync_copy(
                    node_t.at[pl.ds(0, d), pl.ds(n_chunks * _CW, n_tail)],
                    ntail)
                _transpose_block(ntail, ob0, n_tail)
                pltpu.sync_copy(
                    ob0.at[pl.ds(0, n_tail // 2)],
                    node_p.at[pl.ds(n_chunks * (_CW // 2), n_tail // 2)])

        r_full = n_rel // _BLK
        r_tail = n_rel % _BLK

        @pl.when(w == 1)
        def _():
            def rstep(b, _):
                pltpu.sync_copy(
                    rel_t.at[pl.ds(0, d), pl.ds(b * _BLK, _BLK)], rblk)
                _transpose_diag(rblk, ob1, _BLK)
                pltpu.sync_copy(
                    ob1.at[pl.ds(0, _BLK // 2)],
                    rel_p.at[pl.ds(b * (_BLK // 2), _BLK // 2)])
                return 0

            lax.fori_loop(0, r_full, rstep, 0)
            if r_tail:
                pltpu.sync_copy(
                    rel_t.at[pl.ds(0, d), pl.ds(r_full * _BLK, r_tail)],
                    rtail)
                _transpose_block(rtail, ob1, r_tail)
                pltpu.sync_copy(
                    ob1.at[pl.ds(0, r_tail // 2)],
                    rel_p.at[pl.ds(r_full * (_BLK // 2), r_tail // 2)])

    return repack


def _make_gather(n_nodes, n_rel, d, b):
    b_per_w = b // _NW
    n_groups = b_per_w // _L
    n_chunks = b_per_w // _BLK

    @functools.partial(
        pl.kernel,
        mesh=_mesh,
        compiler_params=_params,
        out_type=(
            jax.ShapeDtypeStruct((d, b), jnp.float32),
            jax.ShapeDtypeStruct((d, b), jnp.float32),
            jax.ShapeDtypeStruct((d, b), jnp.float32),
        ),
        scratch_types=[
            pltpu.VMEM((b_per_w,), jnp.int32),
            pltpu.VMEM((b_per_w,), jnp.int32),
            pltpu.VMEM((b_per_w,), jnp.int32),
            pltpu.VMEM((b_per_w,), jnp.int32),
            pltpu.VMEM((b_per_w,), jnp.int32),
            pltpu.VMEM((b_per_w,), jnp.int32),
            pltpu.VMEM((b_per_w, 2 * d), jnp.float32),
            pltpu.VMEM((d, b_per_w), jnp.float32),
            pltpu.SemaphoreType.DMA,
            pltpu.SemaphoreType.DMA,
        ],
    )
    def gather(head_hbm, rel_hbm, tail_hbm, node_p, rel_p,
               head_out, rel_out, tail_out,
               hidx, tidx, ridx, hq, tq, rq,
               pairs, colout, sem_g, sem_w):
        w = _wid()
        base = w * b_per_w
        sl = pl.ds(base, b_per_w)

        pltpu.sync_copy(head_hbm.at[sl], hidx)
        pltpu.sync_copy(tail_hbm.at[sl], tidx)
        pltpu.sync_copy(rel_hbm.at[sl], ridx)

        def compute_q(g, _):
            s = pl.ds(g * _L, _L)
            hq[s] = lax.shift_right_logical(hidx[s], 1)
            tq[s] = lax.shift_right_logical(tidx[s], 1)
            rq[s] = lax.shift_right_logical(ridx[s], 1)
            return 0

        lax.fori_loop(0, n_groups, compute_q, 0)

        first = [True]
        for tab, q, idx, out_hbm in (
            (node_p, hq, hidx, head_out),
            (rel_p, rq, ridx, rel_out),
            (node_p, tq, tidx, tail_out),
        ):
            gathers = [
                pltpu.async_copy(
                    tab.at[q.at[pl.ds(c * _BLK, _BLK)]],
                    pairs.at[pl.ds(c * _BLK, _BLK)], sem_g)
                for c in range(n_chunks)
            ]
            if not first[0]:
                pltpu.make_async_copy(
                    head_out.at[pl.ds(0, d), pl.ds(0, b_per_w)],
                    colout, sem_w).wait()
            first[0] = False
            for g in gathers:
                g.wait()

            iota = lax.iota(jnp.int32, _L)

            @plsc.parallel_loop(0, n_groups, unroll=1)
            def select(g, idx=idx):
                r0 = g * _L
                rowvec = iota + r0
                iv = idx[pl.ds(r0, _L)]
                offv = lax.shift_left(
                    lax.bitwise_and(iv, jnp.int32(1)), 6)
                for j in range(_L):
                    m = jnp.bitwise_and(iota + j, jnp.int32(15))
                    for d0 in range(0, d, _L):
                        fv = m + d0
                        v = plsc.load_gather(pairs, [rowvec, offv + fv])
                        plsc.store_scatter(colout, [fv, rowvec], v)
            pltpu.async_copy(colout, out_hbm.at[pl.ds(0, d), sl], sem_w)
        pltpu.make_async_copy(
            head_out.at[pl.ds(0, d), pl.ds(0, b_per_w)], colout,
            sem_w).wait()

    return gather


def kernel(head_index, rel_type, tail_index, node_emb, rel_emb):
    B = head_index.shape[0]
    N, D = node_emb.shape
    R = rel_emb.shape[0]
    assert D == 64 and B % (_NW * _BLK) == 0

    node_t = node_emb.T
    rel_t = rel_emb.T

    node_p, rel_p = _make_repack(N, R, D)(
        head_index, rel_type, tail_index, node_t, rel_t)
    h, r, t = _make_gather(N, R, D, B)(
        head_index, rel_type, tail_index, node_p, rel_p)
    return (h.T, r.T, t.T)


# speedup vs baseline: 3.2394x; 1.0992x over previous
import functools

import jax
import jax.numpy as jnp
from jax import lax
from jax.experimental import pallas as pl
from jax.experimental.pallas import tpu as pltpu
from jax.experimental.pallas import tpu_sc as plsc

_L = 16
_BLK = 128

_info = plsc.get_sparse_core_info()
_NW = _info.num_cores * _info.num_subcores

_mesh = plsc.VectorSubcoreMesh(core_axis_name="c", subcore_axis_name="s")
_params = pltpu.CompilerParams(
    use_tc_tiling_on_sc=True, needs_layout_passes=False)


def _wid():
    return lax.axis_index("s") * _info.num_cores + lax.axis_index("c")


def _transpose_block(src, dst, width):
    dvecs = [lax.iota(jnp.int32, _L) + g * _L for g in range(4)]
    zero = lax.iota(jnp.int32, _L) * 0

    @plsc.parallel_loop(0, width // 2, unroll=8)
    def body(np_):
        for par in range(2):
            nvec = zero + (np_ * 2 + par)
            for g in range(4):
                v = plsc.load_gather(src, [dvecs[g], nvec])
                dst[np_, pl.ds(par * 64 + g * _L, _L)] = v


def _transpose_diag(src, dst, width):
    iota = lax.iota(jnp.int32, _L)
    dvecs = [iota + d0 for d0 in range(0, 64, _L)]

    @plsc.parallel_loop(0, width // _L, unroll=1)
    def body(g):
        n0 = g * _L
        n0h = lax.shift_right_logical(n0, 1)
        for j in range(_L):
            m = jnp.bitwise_and(iota + j, jnp.int32(15))
            nvec = m + n0
            rowvec = lax.shift_right_logical(m, 1) + n0h
            colbase = lax.shift_left(
                jnp.bitwise_and(m, jnp.int32(1)), 6) + iota
            for gi in range(4):
                v = plsc.load_gather(src, [dvecs[gi], nvec])
                plsc.store_scatter(dst, [rowvec, colbase + gi * _L], v)


_CW = 3 * _BLK


def _make_repack(n_nodes, n_rel, d):
    n_chunks = n_nodes // _CW
    n_tail = n_nodes % _CW
    assert n_tail < _BLK
    per_w = n_chunks // _NW
    rem = n_chunks % _NW

    @functools.partial(
        pl.kernel,
        mesh=_mesh,
        compiler_params=_params,
        out_type=(
            jax.ShapeDtypeStruct((n_nodes // 2, 2 * d), jnp.float32),
            jax.ShapeDtypeStruct((n_rel // 2, 2 * d), jnp.float32),
        ),
        scratch_types=[
            pltpu.VMEM((d, _CW), jnp.float32),
            pltpu.VMEM((d, _CW), jnp.float32),
            pltpu.VMEM((_CW // 2, 2 * d), jnp.float32),
            pltpu.VMEM((_CW // 2, 2 * d), jnp.float32),
            pltpu.VMEM((d, _BLK), jnp.float32),
            pltpu.VMEM((d, n_nodes % _CW or _L), jnp.float32),
            pltpu.VMEM((d, n_rel % _BLK or _L), jnp.float32),
            pltpu.SemaphoreType.DMA,
            pltpu.SemaphoreType.DMA,
            pltpu.SemaphoreType.DMA,
            pltpu.SemaphoreType.DMA,
        ],
    )
    def repack(head_hbm, rel_hbm, tail_hbm, node_t, rel_t,
               node_p, rel_p, in0, in1, ob0, ob1, rblk, ntail, rtail,
               si0, si1, so0, so1):
        w = _wid()
        ins = (in0, in1)
        sis = (si0, si1)
        obs = (ob0, ob1)
        sos = (so0, so1)

        nchk = per_w + jnp.where(w < rem, 1, 0)

        def start_in(k, ph):
            cid = k * _NW + w
            pltpu.async_copy(
                node_t.at[pl.ds(0, d), pl.ds(cid * _CW, _CW)],
                ins[ph], sis[ph])

        @pl.when(nchk > 0)
        def _():
            start_in(0, 0)

        def step(k, _):
            for ph in range(2):
                kk = k * 2 + ph

                @pl.when(kk < nchk)
                def _():
                    @pl.when(kk + 1 < nchk)
                    def _():
                        start_in(kk + 1, 1 - ph)
                    pltpu.make_async_copy(
                        node_t.at[pl.ds(0, d), pl.ds(0, _CW)],
                        ins[ph], sis[ph]).wait()
                    @pl.when(kk >= 2)
                    def _():
                        pltpu.make_async_copy(
                            obs[ph], node_p.at[pl.ds(0, _CW // 2)],
                            sos[ph]).wait()
                    _transpose_diag(ins[ph], obs[ph], _CW)
                    cid = kk * _NW + w
                    pltpu.async_copy(
                        obs[ph],
                        node_p.at[pl.ds(cid * (_CW // 2), _CW // 2)],
                        sos[ph])
            return 0

        lax.fori_loop(0, (per_w + 2) // 2, step, 0)

        for ph in range(2):
            @pl.when(nchk > ph)
            def _(ph=ph):
                pltpu.make_async_copy(
                    obs[ph], node_p.at[pl.ds(0, _CW // 2)],
                    sos[ph]).wait()

        if n_tail:
            @pl.when(w == 0)
            def _():
                pltpu.sync_copy(
                    node_t.at[pl.ds(0, d), pl.ds(n_chunks * _CW, n_tail)],
                    ntail)
                _transpose_block(ntail, ob0, n_tail)
                pltpu.sync_copy(
                    ob0.at[pl.ds(0, n_tail // 2)],
                    node_p.at[pl.ds(n_chunks * (_CW // 2), n_tail // 2)])

        r_full = n_rel // _BLK
        r_tail = n_rel % _BLK

        @pl.when(w == 1)
        def _():
            def rstep(b, _):
                pltpu.sync_copy(
                    rel_t.at[pl.ds(0, d), pl.ds(b * _BLK, _BLK)], rblk)
                _transpose_diag(rblk, ob1, _BLK)
                pltpu.sync_copy(
                    ob1.at[pl.ds(0, _BLK // 2)],
                    rel_p.at[pl.ds(b * (_BLK // 2), _BLK // 2)])
                return 0

            lax.fori_loop(0, r_full, rstep, 0)
            if r_tail:
                pltpu.sync_copy(
                    rel_t.at[pl.ds(0, d), pl.ds(r_full * _BLK, r_tail)],
                    rtail)
                _transpose_block(rtail, ob1, r_tail)
                pltpu.sync_copy(
                    ob1.at[pl.ds(0, r_tail // 2)],
                    rel_p.at[pl.ds(r_full * (_BLK // 2), r_tail // 2)])

    return repack


def _make_gather(n_nodes, n_rel, d, b):
    b_per_w = b // _NW
    n_groups = b_per_w // _L
    n_chunks = b_per_w // _BLK

    @functools.partial(
        pl.kernel,
        mesh=_mesh,
        compiler_params=_params,
        out_type=(
            jax.ShapeDtypeStruct((d, b), jnp.float32),
            jax.ShapeDtypeStruct((d, b), jnp.float32),
            jax.ShapeDtypeStruct((d, b), jnp.float32),
        ),
        scratch_types=[
            pltpu.VMEM((b_per_w,), jnp.int32),
            pltpu.VMEM((b_per_w,), jnp.int32),
            pltpu.VMEM((b_per_w,), jnp.int32),
            pltpu.VMEM((b_per_w,), jnp.int32),
            pltpu.VMEM((b_per_w,), jnp.int32),
            pltpu.VMEM((b_per_w,), jnp.int32),
            pltpu.VMEM((b_per_w, 2 * d), jnp.float32),
            pltpu.VMEM((d, b_per_w), jnp.float32),
            pltpu.SemaphoreType.DMA,
            pltpu.SemaphoreType.DMA,
        ],
    )
    def gather(head_hbm, rel_hbm, tail_hbm, node_p, rel_p,
               head_out, rel_out, tail_out,
               hidx, tidx, ridx, hq, tq, rq,
               pairs, colout, sem_g, sem_w):
        w = _wid()
        base = w * b_per_w
        sl = pl.ds(base, b_per_w)

        pltpu.sync_copy(head_hbm.at[sl], hidx)
        pltpu.sync_copy(tail_hbm.at[sl], tidx)
        pltpu.sync_copy(rel_hbm.at[sl], ridx)

        def compute_q(g, _):
            s = pl.ds(g * _L, _L)
            hq[s] = lax.shift_right_logical(hidx[s], 1)
            tq[s] = lax.shift_right_logical(tidx[s], 1)
            rq[s] = lax.shift_right_logical(ridx[s], 1)
            return 0

        lax.fori_loop(0, n_groups, compute_q, 0)

        first = [True]
        for tab, q, idx, out_hbm in (
            (node_p, hq, hidx, head_out),
            (rel_p, rq, ridx, rel_out),
            (node_p, tq, tidx, tail_out),
        ):
            gathers = [
                pltpu.async_copy(
                    tab.at[q.at[pl.ds(c * _BLK, _BLK)]],
                    pairs.at[pl.ds(c * _BLK, _BLK)], sem_g)
                for c in range(n_chunks)
            ]
            if not first[0]:
                pltpu.make_async_copy(
                    head_out.at[pl.ds(0, d), pl.ds(0, b_per_w)],
                    colout, sem_w).wait()
            first[0] = False
            for g in gathers:
                g.wait()

            iota = lax.iota(jnp.int32, _L)

            @plsc.parallel_loop(0, n_groups, unroll=1)
            def select(g, idx=idx):
                r0 = g * _L
                rowvec = iota + r0
                iv = idx[pl.ds(r0, _L)]
                offv = lax.shift_left(
                    lax.bitwise_and(iv, jnp.int32(1)), 6)
                for j in range(_L):
                    m = jnp.bitwise_and(iota + j, jnp.int32(15))
                    for d0 in range(0, d, _L):
                        fv = m + d0
                        v = plsc.load_gather(pairs, [rowvec, offv + fv])
                        plsc.store_scatter(colout, [fv, rowvec], v)
            pltpu.async_copy(colout, out_hbm.at[pl.ds(0, d), sl], sem_w)
        pltpu.make_async_copy(
            head_out.at[pl.ds(0, d), pl.ds(0, b_per_w)], colout,
            sem_w).wait()

    return gather


def kernel(head_index, rel_type, tail_index, node_emb, rel_emb):
    B = head_index.shape[0]
    N, D = node_emb.shape
    R = rel_emb.shape[0]
    assert D == 64 and B % (_NW * _BLK) == 0

    node_t = node_emb.T
    rel_t = rel_emb.T

    node_p, rel_p = _make_repack(N, R, D)(
        head_index, rel_type, tail_index, node_t, rel_t)
    h, r, t = _make_gather(N, R, D, B)(
        head_index, rel_type, tail_index, node_p, rel_p)
    return (h.T, r.T, t.T)
